# Initial kernel scaffold; baseline (speedup 1.0000x reference)
import jax
import jax.numpy as jnp
from jax.experimental import pallas as pl


def kernel(qrl_token, doc_token, doc_ids, word_embedding, docs_adj, idf_table, params):
    raise NotImplementedError("write your pallas kernel here")


# speedup vs baseline: 1.3097x; 1.3097x over previous
import functools

import jax
import jax.numpy as jnp
from jax import lax
from jax.experimental import pallas as pl
from jax.experimental.pallas import tpu as pltpu
from jax.experimental.pallas import tpu_sc as plsc

B = 64
Q = 5
D = 300
EMB = 300
EMBP = 384
K1 = 240
K2 = 192
TOPK = 40
RPE = 312
TOT = B * RPE
NW = 32
PER_W = TOT // NW
CH = 104
NCH = PER_W // CH
IDF_R = 784
BIG = 1e30


def _sc_gather(table_pad, tok_flat):
  mesh = plsc.VectorSubcoreMesh(core_axis_name="c", subcore_axis_name="s")

  @functools.partial(
      pl.kernel,
      mesh=mesh,
      out_type=jax.ShapeDtypeStruct((TOT, EMBP), jnp.float32),
      scratch_types=[
          pltpu.VMEM((CH,), jnp.int32),
          pltpu.VMEM((CH, EMBP), jnp.float32),
          pltpu.SemaphoreType.DMA,
      ],
  )
  def gather_kernel(table_hbm, tok_hbm, out_hbm, idx_v, rows_v, sem):
    c = lax.axis_index("c")
    s = lax.axis_index("s")
    wid = s * 2 + c
    base = wid * PER_W

    def chunk(i, carry):
      off = base + i * CH
      pltpu.sync_copy(tok_hbm.at[pl.ds(off, CH)], idx_v)
      pltpu.async_copy(table_hbm.at[idx_v], rows_v, sem).wait()
      pltpu.sync_copy(rows_v, out_hbm.at[pl.ds(off, CH)])
      return carry

    lax.fori_loop(0, NCH, chunk, 0)

  return gather_kernel(table_pad, tok_flat)


def _tc_body(ids_ref, qtok_ref, rows_ref, idft_ref, adj_ref, wq_ref, bq_ref,
             sv_ref, pw_ref, pb_ref, l1w_ref, l1b_ref, l2w_ref, l2b_ref,
             l3w_ref, l3b_ref, out_ref):
  f32 = jnp.float32
  qe = rows_ref[0:Q, 0:EMB]
  dm = rows_ref[8:8 + D, 0:EMB]
  A = adj_ref[0]

  def sig(x):
    return 1.0 / (1.0 + jnp.exp(-x))

  dn = jnp.sqrt(jnp.sum(dm * dm, axis=1, keepdims=True)) + 1e-9
  qn = jnp.sqrt(jnp.sum(qe * qe, axis=1, keepdims=True)) + 1e-9
  feat_t = lax.dot_general(qe / qn, dm / dn, (((1,), (1,)), ((), ())),
                           preferred_element_type=f32)

  def amul(x_t):
    return lax.dot_general(x_t, A, (((1,), (1,)), ((), ())),
                           preferred_element_type=f32)

  def lin_t(i, x_t):
    return lax.dot_general(wq_ref[i], x_t, (((0,), (0,)), ((), ())),
                           preferred_element_type=f32) + bq_ref[i]

  def gated_q(i0, x_t, a_t):
    z = sig(lin_t(i0, a_t) + lin_t(i0 + 1, x_t))
    r = sig(lin_t(i0 + 2, a_t) + lin_t(i0 + 3, x_t))
    h = jnp.maximum(lin_t(i0 + 4, a_t) + lin_t(i0 + 5, r * x_t), 0.0)
    return h * z + x_t * (1.0 - z)

  def gated_s(o, x, a):
    sv = lambda j: sv_ref[0, o + j]
    z = sig(sv(0) * a + sv(1) + sv(2) * x + sv(3))
    r = sig(sv(4) * a + sv(5) + sv(6) * x + sv(7))
    h = jnp.maximum(sv(8) * a + sv(9) + sv(10) * (r * x) + sv(11), 0.0)
    return h * z + x * (1.0 - z)

  ei = lax.broadcasted_iota(jnp.int32, (D, D), 0)
  di = lax.broadcasted_iota(jnp.int32, (D, D), 1)
  ident = jnp.where(ei == di, 1.0, 0.0)
  ones_row = jnp.full((1, D), 1.0, f32)

  def topmask(s_row, kk):
    s_col = lax.dot_general(ident, s_row, (((1,), (1,)), ((), ())),
                            preferred_element_type=f32)
    G = jnp.where((s_col > s_row) | ((s_col == s_row) & (ei < di)), 1.0, 0.0)
    rank = lax.dot_general(ones_row, G, (((1,), (0,)), ((), ())),
                           preferred_element_type=f32)
    return rank < (kk - 0.5)

  rep1_t = gated_q(0, feat_t, amul(feat_t))
  x2_t = lax.dot_general(pw_ref[:, 0:1], rep1_t, (((0,), (0,)), ((), ())),
                         preferred_element_type=f32) + pb_ref[0, 0]
  s_t = gated_s(0, x2_t, amul(x2_t))
  mask1 = topmask(s_t, K1)
  t1 = jnp.where(mask1, jnp.tanh(s_t), 0.0)
  xg_t = t1 * rep1_t
  att1m = jnp.where(mask1, xg_t, -BIG)
  rep3_t = gated_q(6, xg_t, amul(xg_t))
  x4_t = lax.dot_general(pw_ref[:, 1:2], rep3_t, (((0,), (0,)), ((), ())),
                         preferred_element_type=f32) + pb_ref[0, 1]
  s4_t = gated_s(12, x4_t, amul(jnp.where(mask1, x4_t, 0.0)))
  s4m = jnp.where(mask1, s4_t, -BIG)
  mask4 = topmask(s4m, K2)
  t4 = jnp.where(mask4, jnp.tanh(s4_t), 0.0)
  att2m = jnp.where(mask4, t4 * rep3_t, -BIG)

  V0 = jnp.concatenate([feat_t, att1m, att2m], axis=0)
  kcol = lax.broadcasted_iota(jnp.int32, (1, TOPK), 1)
  lane = lax.broadcasted_iota(jnp.int32, (15, D), 1)
  big_i = jnp.int32(1 << 30)

  def step(k, carry):
    V, T = carry
    m = jnp.max(V, axis=1, keepdims=True)
    T = T + m * jnp.where(kcol == k, 1.0, 0.0)
    cand = jnp.where(V == m, lane, big_i)
    pmin = jnp.min(cand, axis=1, keepdims=True)
    V = jnp.where(cand == pmin, -BIG, V)
    return (V, T)

  _, T = lax.fori_loop(0, TOPK, step, (V0, jnp.zeros((15, TOPK), f32)))

  attx = jnp.concatenate([T[0:5], T[5:10], T[10:15]], axis=1)
  r1 = jnp.maximum(lax.dot_general(attx, l1w_ref[...], (((1,), (0,)), ((), ())),
                                   preferred_element_type=f32) + l1b_ref[...], 0.0)
  r2 = jnp.maximum(lax.dot_general(r1, l2w_ref[...], (((1,), (0,)), ((), ())),
                                   preferred_element_type=f32) + l2b_ref[...], 0.0)
  r3 = lax.dot_general(r2, l3w_ref[...], (((1,), (0,)), ((), ())),
                       preferred_element_type=f32) + l3b_ref[0, 0]

  b = pl.program_id(0)
  lane128 = lax.broadcasted_iota(jnp.int32, (1, 128), 1)
  q5 = lax.broadcasted_iota(jnp.int32, (1, Q), 1)
  idf = jnp.zeros((1, Q), f32)
  for qi in range(Q):
    tok = qtok_ref[b, qi]
    rowv = idft_ref[pl.ds(tok // 128, 1), :]
    val = jnp.sum(jnp.where(lane128 == tok % 128, rowv, 0.0),
                  axis=1, keepdims=True)
    idf = idf + val * jnp.where(q5 == qi, 1.0, 0.0)
  g = sv_ref[0, 24] * idf + sv_ref[0, 25]
  ge = jnp.exp(g - jnp.max(g, axis=1, keepdims=True))
  gw = ge / jnp.sum(ge, axis=1, keepdims=True)
  res = lax.dot_general(gw, r3, (((1,), (0,)), ((), ())),
                        preferred_element_type=f32)
  out_ref[0] = res


def _pack(p):
  qnames = ['z0', 'z1', 'r0', 'r1', 'h0', 'h1',
            'z03', 'z13', 'r03', 'r13', 'h03', 'h13']
  wq = jnp.stack([p[n + '_w'] for n in qnames])
  bq = jnp.stack([p[n + '_b'] for n in qnames])[:, :, None]

  def spack(suf):
    out = []
    for g in ('z', 'r', 'h'):
      for side in ('0', '1'):
        out.append(p[g + side + suf + '_w'][0, 0])
        out.append(p[g + side + suf + '_b'][0])
    return out

  sv = jnp.stack(spack('2') + spack('4')
                 + [p['gated_w'][0, 0], p['gated_b'][0]]).reshape(1, 26)
  pw = jnp.concatenate([p['p1_w'], p['p2_w']], axis=1)
  pb = jnp.stack([p['p1_b'][0], p['p2_b'][0]]).reshape(1, 2)
  return (wq, bq, sv, pw, pb, p['l1_w'], p['l1_b'].reshape(1, 64),
          p['l2_w'], p['l2_b'].reshape(1, 32), p['l3_w'],
          p['l3_b'].reshape(1, 1))


def _tc_in_specs():
  return [
      pl.BlockSpec((RPE, EMBP), lambda b, ids, qts: (b, 0)),
      pl.BlockSpec((IDF_R, 128), lambda b, ids, qts: (0, 0)),
      pl.BlockSpec((1, D, D), lambda b, ids, qts: (ids[b], 0, 0)),
      pl.BlockSpec((12, 5, 5), lambda b, ids, qts: (0, 0, 0)),
      pl.BlockSpec((12, 5, 1), lambda b, ids, qts: (0, 0, 0)),
      pl.BlockSpec((1, 26), lambda b, ids, qts: (0, 0)),
      pl.BlockSpec((5, 2), lambda b, ids, qts: (0, 0)),
      pl.BlockSpec((1, 2), lambda b, ids, qts: (0, 0)),
      pl.BlockSpec((120, 64), lambda b, ids, qts: (0, 0)),
      pl.BlockSpec((1, 64), lambda b, ids, qts: (0, 0)),
      pl.BlockSpec((64, 32), lambda b, ids, qts: (0, 0)),
      pl.BlockSpec((1, 32), lambda b, ids, qts: (0, 0)),
      pl.BlockSpec((32, 1), lambda b, ids, qts: (0, 0)),
      pl.BlockSpec((1, 1), lambda b, ids, qts: (0, 0)),
  ]


def kernel(qrl_token, doc_token, doc_ids, word_embedding, docs_adj,
           idf_table, params):
  qt = qrl_token.astype(jnp.int32)
  dt = doc_token.astype(jnp.int32)
  ids = doc_ids.astype(jnp.int32)
  fill = (lax.broadcasted_iota(jnp.int32, (B, 7), 0) * 7
          + lax.broadcasted_iota(jnp.int32, (B, 7), 1))
  tok = jnp.concatenate([qt, fill[:, 0:3], dt, fill[:, 3:7]],
                        axis=1).reshape(-1)

  table_pad = jnp.pad(word_embedding, ((0, 0), (0, EMBP - EMB)))
  rows = _sc_gather(table_pad, tok)

  idft = jnp.pad(idf_table, (0, IDF_R * 128 - idf_table.shape[0])
                 ).reshape(IDF_R, 128)

  out = pl.pallas_call(
      _tc_body,
      grid_spec=pltpu.PrefetchScalarGridSpec(
          num_scalar_prefetch=2,
          grid=(B,),
          in_specs=_tc_in_specs(),
          out_specs=pl.BlockSpec((1, 1, 1), lambda b, ids, qts: (b, 0, 0)),
      ),
      out_shape=jax.ShapeDtypeStruct((B, 1, 1), jnp.float32),
  )(ids, qt, rows, idft, docs_adj, *_pack(params))
  return out.reshape(B, 1)


# speedup vs baseline: 1.7147x; 1.3093x over previous
import functools

import jax
import jax.numpy as jnp
from jax import lax
from jax.experimental import pallas as pl
from jax.experimental.pallas import tpu as pltpu
from jax.experimental.pallas import tpu_sc as plsc

B = 64
Q = 5
D = 300
EMB = 300
EMBP = 384
PAD_R = 2000
NROW = 100000
K1 = 240
K2 = 192
TOPK = 40
RPE = 312
TOT = B * RPE
NW = 32
PER_W = TOT // NW
CH = 104
NCH = PER_W // CH
IDF_R = 784
BIG = 1e30


def _sc_gather(table_pad, tok_flat):
  mesh = plsc.VectorSubcoreMesh(core_axis_name="c", subcore_axis_name="s")

  @functools.partial(
      pl.kernel,
      mesh=mesh,
      out_type=jax.ShapeDtypeStruct((TOT, EMBP), jnp.float32),
      scratch_types=[
          pltpu.VMEM((CH,), jnp.int32),
          pltpu.VMEM((CH, EMBP), jnp.float32),
          pltpu.SemaphoreType.DMA,
      ],
  )
  def gather_kernel(table_hbm, tok_hbm, out_hbm, idx_v, rows_v, sem):
    c = lax.axis_index("c")
    s = lax.axis_index("s")
    wid = s * 2 + c
    base = wid * PER_W

    def chunk(i, carry):
      off = base + i * CH
      pltpu.sync_copy(tok_hbm.at[pl.ds(off, CH)], idx_v)
      pltpu.async_copy(table_hbm.at[idx_v], rows_v, sem).wait()
      pltpu.sync_copy(rows_v, out_hbm.at[pl.ds(off, CH)])
      return carry

    lax.fori_loop(0, NCH, chunk, 0)

  return gather_kernel(table_pad, tok_flat)


def _tc_body(ids_ref, qtok_ref, rows_ref, idft_ref, adj_ref, wq_ref, bq_ref,
             sv_ref, pw_ref, pb_ref, l1w_ref, l1b_ref, l2w_ref, l2b_ref,
             l3w_ref, l3b_ref, out_ref):
  f32 = jnp.float32
  qe = rows_ref[0:Q, 0:EMB]
  dm = rows_ref[8:8 + D, 0:EMB]
  A = adj_ref[0]

  def sig(x):
    return 1.0 / (1.0 + jnp.exp(-x))

  dn = jnp.sqrt(jnp.sum(dm * dm, axis=1, keepdims=True)) + 1e-9
  qn = jnp.sqrt(jnp.sum(qe * qe, axis=1, keepdims=True)) + 1e-9
  feat_t = lax.dot_general(qe / qn, dm / dn, (((1,), (1,)), ((), ())),
                           preferred_element_type=f32)

  def amul(x_t):
    return lax.dot_general(x_t, A, (((1,), (1,)), ((), ())),
                           preferred_element_type=f32)

  def lin_t(i, x_t):
    return lax.dot_general(wq_ref[i], x_t, (((0,), (0,)), ((), ())),
                           preferred_element_type=f32) + bq_ref[i]

  def gated_q(i0, x_t, a_t):
    z = sig(lin_t(i0, a_t) + lin_t(i0 + 1, x_t))
    r = sig(lin_t(i0 + 2, a_t) + lin_t(i0 + 3, x_t))
    h = jnp.maximum(lin_t(i0 + 4, a_t) + lin_t(i0 + 5, r * x_t), 0.0)
    return h * z + x_t * (1.0 - z)

  def gated_s(o, x, a):
    sv = lambda j: sv_ref[0, o + j]
    z = sig(sv(0) * a + sv(1) + sv(2) * x + sv(3))
    r = sig(sv(4) * a + sv(5) + sv(6) * x + sv(7))
    h = jnp.maximum(sv(8) * a + sv(9) + sv(10) * (r * x) + sv(11), 0.0)
    return h * z + x * (1.0 - z)

  ei = lax.broadcasted_iota(jnp.int32, (D, D), 0)
  di = lax.broadcasted_iota(jnp.int32, (D, D), 1)
  ident = jnp.where(ei == di, 1.0, 0.0)
  ones_row = jnp.full((1, D), 1.0, f32)

  def topmask(s_row, kk):
    s_col = lax.dot_general(ident, s_row, (((1,), (1,)), ((), ())),
                            preferred_element_type=f32)
    G = jnp.where((s_col > s_row) | ((s_col == s_row) & (ei < di)), 1.0, 0.0)
    rank = lax.dot_general(ones_row, G, (((1,), (0,)), ((), ())),
                           preferred_element_type=f32)
    return rank < (kk - 0.5)

  rep1_t = gated_q(0, feat_t, amul(feat_t))
  x2_t = lax.dot_general(pw_ref[:, 0:1], rep1_t, (((0,), (0,)), ((), ())),
                         preferred_element_type=f32) + pb_ref[0, 0]
  s_t = gated_s(0, x2_t, amul(x2_t))
  mask1 = topmask(s_t, K1)
  t1 = jnp.where(mask1, jnp.tanh(s_t), 0.0)
  xg_t = t1 * rep1_t
  att1m = jnp.where(mask1, xg_t, -BIG)
  rep3_t = gated_q(6, xg_t, amul(xg_t))
  x4_t = lax.dot_general(pw_ref[:, 1:2], rep3_t, (((0,), (0,)), ((), ())),
                         preferred_element_type=f32) + pb_ref[0, 1]
  s4_t = gated_s(12, x4_t, amul(jnp.where(mask1, x4_t, 0.0)))
  s4m = jnp.where(mask1, s4_t, -BIG)
  mask4 = topmask(s4m, K2)
  t4 = jnp.where(mask4, jnp.tanh(s4_t), 0.0)
  att2m = jnp.where(mask4, t4 * rep3_t, -BIG)

  V0 = jnp.concatenate([feat_t, att1m, att2m], axis=0)
  kcol = lax.broadcasted_iota(jnp.int32, (1, TOPK), 1)
  lane = lax.broadcasted_iota(jnp.int32, (15, D), 1)
  big_i = jnp.int32(1 << 30)

  def step(k, carry):
    V, T = carry
    m = jnp.max(V, axis=1, keepdims=True)
    T = T + m * jnp.where(kcol == k, 1.0, 0.0)
    cand = jnp.where(V == m, lane, big_i)
    pmin = jnp.min(cand, axis=1, keepdims=True)
    V = jnp.where(cand == pmin, -BIG, V)
    return (V, T)

  _, T = lax.fori_loop(0, TOPK, step, (V0, jnp.zeros((15, TOPK), f32)))

  attx = jnp.concatenate([T[0:5], T[5:10], T[10:15]], axis=1)
  r1 = jnp.maximum(lax.dot_general(attx, l1w_ref[...], (((1,), (0,)), ((), ())),
                                   preferred_element_type=f32) + l1b_ref[...], 0.0)
  r2 = jnp.maximum(lax.dot_general(r1, l2w_ref[...], (((1,), (0,)), ((), ())),
                                   preferred_element_type=f32) + l2b_ref[...], 0.0)
  r3 = lax.dot_general(r2, l3w_ref[...], (((1,), (0,)), ((), ())),
                       preferred_element_type=f32) + l3b_ref[0, 0]

  b = pl.program_id(0)
  lane128 = lax.broadcasted_iota(jnp.int32, (1, 128), 1)
  q5 = lax.broadcasted_iota(jnp.int32, (1, Q), 1)
  idf = jnp.zeros((1, Q), f32)
  for qi in range(Q):
    tok = qtok_ref[b, qi]
    rowv = idft_ref[pl.ds(tok // 128, 1), :]
    val = jnp.sum(jnp.where(lane128 == tok % 128, rowv, 0.0),
                  axis=1, keepdims=True)
    idf = idf + val * jnp.where(q5 == qi, 1.0, 0.0)
  g = sv_ref[0, 24] * idf + sv_ref[0, 25]
  ge = jnp.exp(g - jnp.max(g, axis=1, keepdims=True))
  gw = ge / jnp.sum(ge, axis=1, keepdims=True)
  res = lax.dot_general(gw, r3, (((1,), (0,)), ((), ())),
                        preferred_element_type=f32)
  out_ref[0] = res


def _pack(p):
  qnames = ['z0', 'z1', 'r0', 'r1', 'h0', 'h1',
            'z03', 'z13', 'r03', 'r13', 'h03', 'h13']
  wq = jnp.stack([p[n + '_w'] for n in qnames])
  bq = jnp.stack([p[n + '_b'] for n in qnames])[:, :, None]

  def spack(suf):
    out = []
    for g in ('z', 'r', 'h'):
      for side in ('0', '1'):
        out.append(p[g + side + suf + '_w'][0, 0])
        out.append(p[g + side + suf + '_b'][0])
    return out

  sv = jnp.stack(spack('2') + spack('4')
                 + [p['gated_w'][0, 0], p['gated_b'][0]]).reshape(1, 26)
  pw = jnp.concatenate([p['p1_w'], p['p2_w']], axis=1)
  pb = jnp.stack([p['p1_b'][0], p['p2_b'][0]]).reshape(1, 2)
  return (wq, bq, sv, pw, pb, p['l1_w'], p['l1_b'].reshape(1, 64),
          p['l2_w'], p['l2_b'].reshape(1, 32), p['l3_w'],
          p['l3_b'].reshape(1, 1))


def _tc_in_specs():
  return [
      pl.BlockSpec((RPE, EMBP), lambda b, ids, qts: (b, 0)),
      pl.BlockSpec((IDF_R, 128), lambda b, ids, qts: (0, 0)),
      pl.BlockSpec((1, D, D), lambda b, ids, qts: (ids[b], 0, 0)),
      pl.BlockSpec((12, 5, 5), lambda b, ids, qts: (0, 0, 0)),
      pl.BlockSpec((12, 5, 1), lambda b, ids, qts: (0, 0, 0)),
      pl.BlockSpec((1, 26), lambda b, ids, qts: (0, 0)),
      pl.BlockSpec((5, 2), lambda b, ids, qts: (0, 0)),
      pl.BlockSpec((1, 2), lambda b, ids, qts: (0, 0)),
      pl.BlockSpec((120, 64), lambda b, ids, qts: (0, 0)),
      pl.BlockSpec((1, 64), lambda b, ids, qts: (0, 0)),
      pl.BlockSpec((64, 32), lambda b, ids, qts: (0, 0)),
      pl.BlockSpec((1, 32), lambda b, ids, qts: (0, 0)),
      pl.BlockSpec((32, 1), lambda b, ids, qts: (0, 0)),
      pl.BlockSpec((1, 1), lambda b, ids, qts: (0, 0)),
  ]


def kernel(qrl_token, doc_token, doc_ids, word_embedding, docs_adj,
           idf_table, params):
  qt = qrl_token.astype(jnp.int32)
  dt = doc_token.astype(jnp.int32)
  ids = doc_ids.astype(jnp.int32)
  fill = (lax.broadcasted_iota(jnp.int32, (B, 7), 0) * 7
          + lax.broadcasted_iota(jnp.int32, (B, 7), 1))
  tok = jnp.concatenate([qt, fill[:, 0:3], dt, fill[:, 3:7]],
                        axis=1).reshape(-1)

  def _pad_body(src_ref, dst_ref):
    dst_ref[:, 0:EMB] = src_ref[...]

  table_pad = pl.pallas_call(
      _pad_body,
      grid=(NROW // PAD_R,),
      in_specs=[pl.BlockSpec((PAD_R, EMB), lambda i: (i, 0))],
      out_specs=pl.BlockSpec((PAD_R, EMBP), lambda i: (i, 0)),
      out_shape=jax.ShapeDtypeStruct((NROW, EMBP), jnp.float32),
  )(word_embedding)
  rows = _sc_gather(table_pad, tok)

  idft = jnp.pad(idf_table, (0, IDF_R * 128 - idf_table.shape[0])
                 ).reshape(IDF_R, 128)

  out = pl.pallas_call(
      _tc_body,
      grid_spec=pltpu.PrefetchScalarGridSpec(
          num_scalar_prefetch=2,
          grid=(B,),
          in_specs=_tc_in_specs(),
          out_specs=pl.BlockSpec((1, 1, 1), lambda b, ids, qts: (b, 0, 0)),
      ),
      out_shape=jax.ShapeDtypeStruct((B, 1, 1), jnp.float32),
  )(ids, qt, rows, idft, docs_adj, *_pack(params))
  return out.reshape(B, 1)


# speedup vs baseline: 3.0363x; 1.7707x over previous
import functools

import jax
import jax.numpy as jnp
from jax import lax
from jax.experimental import pallas as pl
from jax.experimental.pallas import tpu as pltpu
from jax.experimental.pallas import tpu_sc as plsc

B = 64
Q = 5
D = 300
EMB = 300
EMBP = 384
PAD_R = 2000
NROW = 100000
K1 = 240
K2 = 192
TOPK = 40
RPE = 312
TOT = B * RPE
NW = 32
PER_W = TOT // NW
CH = 104
NCH = PER_W // CH
IDF_R = 784
G = 8
NB = B // G
BIG = 1e30


def _sc_gather(table_pad, tok_flat):
  mesh = plsc.VectorSubcoreMesh(core_axis_name="c", subcore_axis_name="s")

  @functools.partial(
      pl.kernel,
      mesh=mesh,
      out_type=jax.ShapeDtypeStruct((TOT, EMBP), jnp.float32),
      scratch_types=[
          pltpu.VMEM((CH,), jnp.int32),
          pltpu.VMEM((CH, EMBP), jnp.float32),
          pltpu.SemaphoreType.DMA,
      ],
  )
  def gather_kernel(table_hbm, tok_hbm, out_hbm, idx_v, rows_v, sem):
    c = lax.axis_index("c")
    s = lax.axis_index("s")
    wid = s * 2 + c
    base = wid * PER_W

    def chunk(i, carry):
      off = base + i * CH
      pltpu.sync_copy(tok_hbm.at[pl.ds(off, CH)], idx_v)
      pltpu.async_copy(table_hbm.at[idx_v], rows_v, sem).wait()
      pltpu.sync_copy(rows_v, out_hbm.at[pl.ds(off, CH)])
      return carry

    lax.fori_loop(0, NCH, chunk, 0)

  return gather_kernel(table_pad, tok_flat)


def _tc_body(ids_ref, qtok_ref, rows_ref, idft_ref, adj_hbm, wq_ref, bq_ref,
             sv_ref, pw_ref, pb_ref, l1w_ref, l1b_ref, l2w_ref, l2b_ref,
             l3w_ref, l3b_ref, out_ref, adjbuf, sem):
  f32 = jnp.float32
  i = pl.program_id(0)

  copies = []
  for g in range(G):
    cp = pltpu.make_async_copy(adj_hbm.at[ids_ref[i * G + g]],
                               adjbuf.at[g], sem.at[g])
    cp.start()
    copies.append(cp)

  def sig(x):
    return 1.0 / (1.0 + jnp.exp(-x))

  def lin_t(idx, x_t):
    return lax.dot_general(wq_ref[idx], x_t, (((0,), (0,)), ((), ())),
                           preferred_element_type=f32) + bq_ref[idx]

  def gated_q(i0, x_t, a_t):
    z = sig(lin_t(i0, a_t) + lin_t(i0 + 1, x_t))
    r = sig(lin_t(i0 + 2, a_t) + lin_t(i0 + 3, x_t))
    h = jnp.maximum(lin_t(i0 + 4, a_t) + lin_t(i0 + 5, r * x_t), 0.0)
    return h * z + x_t * (1.0 - z)

  def gated_s(o, x, a):
    sv = lambda j: sv_ref[0, o + j]
    z = sig(sv(0) * a + sv(1) + sv(2) * x + sv(3))
    r = sig(sv(4) * a + sv(5) + sv(6) * x + sv(7))
    h = jnp.maximum(sv(8) * a + sv(9) + sv(10) * (r * x) + sv(11), 0.0)
    return h * z + x * (1.0 - z)

  ei = lax.broadcasted_iota(jnp.int32, (D, D), 0)
  di = lax.broadcasted_iota(jnp.int32, (D, D), 1)
  ident = jnp.where(ei == di, 1.0, 0.0)
  ones_row = jnp.full((1, D), 1.0, f32)

  def topmask(s_row, kk):
    s_col = lax.dot_general(ident, s_row, (((1,), (1,)), ((), ())),
                            preferred_element_type=f32)
    Gm = jnp.where((s_col > s_row) | ((s_col == s_row) & (ei < di)), 1.0, 0.0)
    rank = lax.dot_general(ones_row, Gm, (((1,), (0,)), ((), ())),
                           preferred_element_type=f32)
    return rank < (kk - 0.5)

  V0s = []
  for g in range(G):
    qe = rows_ref[g * RPE + 0:g * RPE + Q, 0:EMB]
    dm = rows_ref[g * RPE + 8:g * RPE + 8 + D, 0:EMB]
    copies[g].wait()
    A = adjbuf[g]

    def amul(x_t):
      return lax.dot_general(x_t, A, (((1,), (1,)), ((), ())),
                             preferred_element_type=f32)

    dn = jnp.sqrt(jnp.sum(dm * dm, axis=1, keepdims=True)) + 1e-9
    qn = jnp.sqrt(jnp.sum(qe * qe, axis=1, keepdims=True)) + 1e-9
    feat_t = lax.dot_general(qe / qn, dm / dn, (((1,), (1,)), ((), ())),
                             preferred_element_type=f32)

    rep1_t = gated_q(0, feat_t, amul(feat_t))
    x2_t = lax.dot_general(pw_ref[:, 0:1], rep1_t, (((0,), (0,)), ((), ())),
                           preferred_element_type=f32) + pb_ref[0, 0]
    s_t = gated_s(0, x2_t, amul(x2_t))
    mask1 = topmask(s_t, K1)
    t1 = jnp.where(mask1, jnp.tanh(s_t), 0.0)
    xg_t = t1 * rep1_t
    att1m = jnp.where(mask1, xg_t, -BIG)
    rep3_t = gated_q(6, xg_t, amul(xg_t))
    x4_t = lax.dot_general(pw_ref[:, 1:2], rep3_t, (((0,), (0,)), ((), ())),
                           preferred_element_type=f32) + pb_ref[0, 1]
    s4_t = gated_s(12, x4_t, amul(jnp.where(mask1, x4_t, 0.0)))
    s4m = jnp.where(mask1, s4_t, -BIG)
    mask4 = topmask(s4m, K2)
    t4 = jnp.where(mask4, jnp.tanh(s4_t), 0.0)
    att2m = jnp.where(mask4, t4 * rep3_t, -BIG)
    V0s.append(feat_t)
    V0s.append(att1m)
    V0s.append(att2m)

  V0 = jnp.concatenate(V0s, axis=0)
  NR = 15 * G
  kcol = lax.broadcasted_iota(jnp.int32, (1, TOPK), 1)
  lane = lax.broadcasted_iota(jnp.int32, (NR, D), 1)
  big_i = jnp.int32(1 << 30)

  def step(k, carry):
    V, T = carry
    m = jnp.max(V, axis=1, keepdims=True)
    T = T + m * jnp.where(kcol == k, 1.0, 0.0)
    cand = jnp.where(V == m, lane, big_i)
    pmin = jnp.min(cand, axis=1, keepdims=True)
    V = jnp.where(cand == pmin, -BIG, V)
    return (V, T)

  _, T = lax.fori_loop(0, TOPK, step, (V0, jnp.zeros((NR, TOPK), f32)))

  lane128 = lax.broadcasted_iota(jnp.int32, (1, 128), 1)
  q5 = lax.broadcasted_iota(jnp.int32, (1, Q), 1)

  for g in range(G):
    Tg = T[15 * g:15 * g + 15]
    attx = jnp.concatenate([Tg[0:5], Tg[5:10], Tg[10:15]], axis=1)
    r1 = jnp.maximum(
        lax.dot_general(attx, l1w_ref[...], (((1,), (0,)), ((), ())),
                        preferred_element_type=f32) + l1b_ref[...], 0.0)
    r2 = jnp.maximum(
        lax.dot_general(r1, l2w_ref[...], (((1,), (0,)), ((), ())),
                        preferred_element_type=f32) + l2b_ref[...], 0.0)
    r3 = lax.dot_general(r2, l3w_ref[...], (((1,), (0,)), ((), ())),
                         preferred_element_type=f32) + l3b_ref[0, 0]

    b = i * G + g
    idf = jnp.zeros((1, Q), f32)
    for qi in range(Q):
      tok = qtok_ref[b, qi]
      rowv = idft_ref[pl.ds(tok // 128, 1), :]
      val = jnp.sum(jnp.where(lane128 == tok % 128, rowv, 0.0),
                    axis=1, keepdims=True)
      idf = idf + val * jnp.where(q5 == qi, 1.0, 0.0)
    gv = sv_ref[0, 24] * idf + sv_ref[0, 25]
    ge = jnp.exp(gv - jnp.max(gv, axis=1, keepdims=True))
    gw = ge / jnp.sum(ge, axis=1, keepdims=True)
    res = lax.dot_general(gw, r3, (((1,), (0,)), ((), ())),
                          preferred_element_type=f32)
    out_ref[g] = res


def _pack(p):
  qnames = ['z0', 'z1', 'r0', 'r1', 'h0', 'h1',
            'z03', 'z13', 'r03', 'r13', 'h03', 'h13']
  wq = jnp.stack([p[n + '_w'] for n in qnames])
  bq = jnp.stack([p[n + '_b'] for n in qnames])[:, :, None]

  def spack(suf):
    out = []
    for gz in ('z', 'r', 'h'):
      for side in ('0', '1'):
        out.append(p[gz + side + suf + '_w'][0, 0])
        out.append(p[gz + side + suf + '_b'][0])
    return out

  sv = jnp.stack(spack('2') + spack('4')
                 + [p['gated_w'][0, 0], p['gated_b'][0]]).reshape(1, 26)
  pw = jnp.concatenate([p['p1_w'], p['p2_w']], axis=1)
  pb = jnp.stack([p['p1_b'][0], p['p2_b'][0]]).reshape(1, 2)
  return (wq, bq, sv, pw, pb, p['l1_w'], p['l1_b'].reshape(1, 64),
          p['l2_w'], p['l2_b'].reshape(1, 32), p['l3_w'],
          p['l3_b'].reshape(1, 1))


def _tc_in_specs():
  return [
      pl.BlockSpec((G * RPE, EMBP), lambda i, ids, qts: (i, 0)),
      pl.BlockSpec((IDF_R, 128), lambda i, ids, qts: (0, 0)),
      pl.BlockSpec(memory_space=pltpu.MemorySpace.HBM),
      pl.BlockSpec((12, 5, 5), lambda i, ids, qts: (0, 0, 0)),
      pl.BlockSpec((12, 5, 1), lambda i, ids, qts: (0, 0, 0)),
      pl.BlockSpec((1, 26), lambda i, ids, qts: (0, 0)),
      pl.BlockSpec((5, 2), lambda i, ids, qts: (0, 0)),
      pl.BlockSpec((1, 2), lambda i, ids, qts: (0, 0)),
      pl.BlockSpec((120, 64), lambda i, ids, qts: (0, 0)),
      pl.BlockSpec((1, 64), lambda i, ids, qts: (0, 0)),
      pl.BlockSpec((64, 32), lambda i, ids, qts: (0, 0)),
      pl.BlockSpec((1, 32), lambda i, ids, qts: (0, 0)),
      pl.BlockSpec((32, 1), lambda i, ids, qts: (0, 0)),
      pl.BlockSpec((1, 1), lambda i, ids, qts: (0, 0)),
  ]


def kernel(qrl_token, doc_token, doc_ids, word_embedding, docs_adj,
           idf_table, params):
  qt = qrl_token.astype(jnp.int32)
  dt = doc_token.astype(jnp.int32)
  ids = doc_ids.astype(jnp.int32)
  fill = (lax.broadcasted_iota(jnp.int32, (B, 7), 0) * 7
          + lax.broadcasted_iota(jnp.int32, (B, 7), 1))
  tok = jnp.concatenate([qt, fill[:, 0:3], dt, fill[:, 3:7]],
                        axis=1).reshape(-1)

  def _pad_body(src_ref, dst_ref):
    dst_ref[:, 0:EMB] = src_ref[...]

  table_pad = pl.pallas_call(
      _pad_body,
      grid=(NROW // PAD_R,),
      in_specs=[pl.BlockSpec((PAD_R, EMB), lambda i: (i, 0))],
      out_specs=pl.BlockSpec((PAD_R, EMBP), lambda i: (i, 0)),
      out_shape=jax.ShapeDtypeStruct((NROW, EMBP), jnp.float32),
  )(word_embedding)
  rows = _sc_gather(table_pad, tok)

  idft = jnp.pad(idf_table, (0, IDF_R * 128 - idf_table.shape[0])
                 ).reshape(IDF_R, 128)

  out = pl.pallas_call(
      _tc_body,
      grid_spec=pltpu.PrefetchScalarGridSpec(
          num_scalar_prefetch=2,
          grid=(NB,),
          in_specs=_tc_in_specs(),
          out_specs=pl.BlockSpec((G, 1, 1), lambda i, ids, qts: (i, 0, 0)),
          scratch_shapes=[
              pltpu.VMEM((G, D, D), jnp.float32),
              pltpu.SemaphoreType.DMA((G,)),
          ],
      ),
      out_shape=jax.ShapeDtypeStruct((B, 1, 1), jnp.float32),
  )(ids, qt, rows, idft, docs_adj, *_pack(params))
  return out.reshape(B, 1)


# speedup vs baseline: 3.2520x; 1.0710x over previous
import functools

import jax
import jax.numpy as jnp
from jax import lax
from jax.experimental import pallas as pl
from jax.experimental.pallas import tpu as pltpu
from jax.experimental.pallas import tpu_sc as plsc

B = 64
Q = 5
D = 300
EMB = 300
EMBP = 384
PAD_R = 5000
NROW = 100000
K1 = 240
K2 = 192
TOPK = 40
RPE = 312
TOT = B * RPE
NW = 32
PER_W = TOT // NW
CH = 104
NCH = PER_W // CH
IDF_R = 784
G = 16
NB = B // G
BIG = 1e30


def _sc_gather(table_pad, tok_flat):
  mesh = plsc.VectorSubcoreMesh(core_axis_name="c", subcore_axis_name="s")

  @functools.partial(
      pl.kernel,
      mesh=mesh,
      out_type=jax.ShapeDtypeStruct((TOT, EMBP), jnp.float32),
      scratch_types=[
          pltpu.VMEM((CH,), jnp.int32),
          pltpu.VMEM((CH, EMBP), jnp.float32),
          pltpu.SemaphoreType.DMA,
      ],
  )
  def gather_kernel(table_hbm, tok_hbm, out_hbm, idx_v, rows_v, sem):
    c = lax.axis_index("c")
    s = lax.axis_index("s")
    wid = s * 2 + c
    base = wid * PER_W

    def chunk(i, carry):
      off = base + i * CH
      pltpu.sync_copy(tok_hbm.at[pl.ds(off, CH)], idx_v)
      pltpu.async_copy(table_hbm.at[idx_v], rows_v, sem).wait()
      pltpu.sync_copy(rows_v, out_hbm.at[pl.ds(off, CH)])
      return carry

    lax.fori_loop(0, NCH, chunk, 0)

  return gather_kernel(table_pad, tok_flat)


def _tc_body(ids_ref, qtok_ref, rows_ref, idft_ref, adj_hbm, wq_ref, bq_ref,
             sv_ref, pw_ref, pb_ref, l1w_ref, l1b_ref, l2w_ref, l2b_ref,
             l3w_ref, l3b_ref, out_ref, adjbuf, sem):
  f32 = jnp.float32
  i = pl.program_id(0)

  copies = []
  for g in range(G):
    cp = pltpu.make_async_copy(adj_hbm.at[ids_ref[i * G + g]],
                               adjbuf.at[g], sem.at[g])
    cp.start()
    copies.append(cp)

  def sig(x):
    return 1.0 / (1.0 + jnp.exp(-x))

  def lin_t(idx, x_t):
    return lax.dot_general(wq_ref[idx], x_t, (((0,), (0,)), ((), ())),
                           preferred_element_type=f32) + bq_ref[idx]

  def gated_q(i0, x_t, a_t):
    z = sig(lin_t(i0, a_t) + lin_t(i0 + 1, x_t))
    r = sig(lin_t(i0 + 2, a_t) + lin_t(i0 + 3, x_t))
    h = jnp.maximum(lin_t(i0 + 4, a_t) + lin_t(i0 + 5, r * x_t), 0.0)
    return h * z + x_t * (1.0 - z)

  def gated_s(o, x, a):
    sv = lambda j: sv_ref[0, o + j]
    z = sig(sv(0) * a + sv(1) + sv(2) * x + sv(3))
    r = sig(sv(4) * a + sv(5) + sv(6) * x + sv(7))
    h = jnp.maximum(sv(8) * a + sv(9) + sv(10) * (r * x) + sv(11), 0.0)
    return h * z + x * (1.0 - z)

  ei = lax.broadcasted_iota(jnp.int32, (D, D), 0)
  di = lax.broadcasted_iota(jnp.int32, (D, D), 1)
  ident = jnp.where(ei == di, 1.0, 0.0)
  ones_row = jnp.full((1, D), 1.0, f32)

  def topmask(s_row, kk):
    s_col = lax.dot_general(ident, s_row, (((1,), (1,)), ((), ())),
                            preferred_element_type=f32)
    Gm = jnp.where((s_col > s_row) | ((s_col == s_row) & (ei < di)), 1.0, 0.0)
    rank = lax.dot_general(ones_row, Gm, (((1,), (0,)), ((), ())),
                           preferred_element_type=f32)
    return rank < (kk - 0.5)

  V0s = []
  for g in range(G):
    qe = rows_ref[g * RPE + 0:g * RPE + Q, 0:EMB]
    dm = rows_ref[g * RPE + 8:g * RPE + 8 + D, 0:EMB]
    copies[g].wait()
    A = adjbuf[g]

    def amul(x_t):
      return lax.dot_general(x_t, A, (((1,), (1,)), ((), ())),
                             preferred_element_type=f32)

    dn = jnp.sqrt(jnp.sum(dm * dm, axis=1, keepdims=True)) + 1e-9
    qn = jnp.sqrt(jnp.sum(qe * qe, axis=1, keepdims=True)) + 1e-9
    feat_t = lax.dot_general(qe / qn, dm / dn, (((1,), (1,)), ((), ())),
                             preferred_element_type=f32)

    rep1_t = gated_q(0, feat_t, amul(feat_t))
    x2_t = lax.dot_general(pw_ref[:, 0:1], rep1_t, (((0,), (0,)), ((), ())),
                           preferred_element_type=f32) + pb_ref[0, 0]
    s_t = gated_s(0, x2_t, amul(x2_t))
    mask1 = topmask(s_t, K1)
    t1 = jnp.where(mask1, jnp.tanh(s_t), 0.0)
    xg_t = t1 * rep1_t
    att1m = jnp.where(mask1, xg_t, -BIG)
    rep3_t = gated_q(6, xg_t, amul(xg_t))
    x4_t = lax.dot_general(pw_ref[:, 1:2], rep3_t, (((0,), (0,)), ((), ())),
                           preferred_element_type=f32) + pb_ref[0, 1]
    s4_t = gated_s(12, x4_t, amul(jnp.where(mask1, x4_t, 0.0)))
    s4m = jnp.where(mask1, s4_t, -BIG)
    mask4 = topmask(s4m, K2)
    t4 = jnp.where(mask4, jnp.tanh(s4_t), 0.0)
    att2m = jnp.where(mask4, t4 * rep3_t, -BIG)
    V0s.append(feat_t)
    V0s.append(att1m)
    V0s.append(att2m)

  V0 = jnp.concatenate(V0s, axis=0)
  NR = 15 * G
  kcol = lax.broadcasted_iota(jnp.int32, (1, TOPK), 1)
  lane = lax.broadcasted_iota(jnp.int32, (NR, D), 1)
  big_i = jnp.int32(1 << 30)

  def step(k, carry):
    V, T = carry
    m = jnp.max(V, axis=1, keepdims=True)
    T = T + m * jnp.where(kcol == k, 1.0, 0.0)
    cand = jnp.where(V == m, lane, big_i)
    pmin = jnp.min(cand, axis=1, keepdims=True)
    V = jnp.where(cand == pmin, -BIG, V)
    return (V, T)

  _, T = lax.fori_loop(0, TOPK, step, (V0, jnp.zeros((NR, TOPK), f32)))

  lane128 = lax.broadcasted_iota(jnp.int32, (1, 128), 1)
  q5 = lax.broadcasted_iota(jnp.int32, (1, Q), 1)

  for g in range(G):
    Tg = T[15 * g:15 * g + 15]
    attx = jnp.concatenate([Tg[0:5], Tg[5:10], Tg[10:15]], axis=1)
    r1 = jnp.maximum(
        lax.dot_general(attx, l1w_ref[...], (((1,), (0,)), ((), ())),
                        preferred_element_type=f32) + l1b_ref[...], 0.0)
    r2 = jnp.maximum(
        lax.dot_general(r1, l2w_ref[...], (((1,), (0,)), ((), ())),
                        preferred_element_type=f32) + l2b_ref[...], 0.0)
    r3 = lax.dot_general(r2, l3w_ref[...], (((1,), (0,)), ((), ())),
                         preferred_element_type=f32) + l3b_ref[0, 0]

    b = i * G + g
    idf = jnp.zeros((1, Q), f32)
    for qi in range(Q):
      tok = qtok_ref[b, qi]
      rowv = idft_ref[pl.ds(tok // 128, 1), :]
      val = jnp.sum(jnp.where(lane128 == tok % 128, rowv, 0.0),
                    axis=1, keepdims=True)
      idf = idf + val * jnp.where(q5 == qi, 1.0, 0.0)
    gv = sv_ref[0, 24] * idf + sv_ref[0, 25]
    ge = jnp.exp(gv - jnp.max(gv, axis=1, keepdims=True))
    gw = ge / jnp.sum(ge, axis=1, keepdims=True)
    res = lax.dot_general(gw, r3, (((1,), (0,)), ((), ())),
                          preferred_element_type=f32)
    out_ref[g] = res


def _pack(p):
  qnames = ['z0', 'z1', 'r0', 'r1', 'h0', 'h1',
            'z03', 'z13', 'r03', 'r13', 'h03', 'h13']
  wq = jnp.stack([p[n + '_w'] for n in qnames])
  bq = jnp.stack([p[n + '_b'] for n in qnames])[:, :, None]

  def spack(suf):
    out = []
    for gz in ('z', 'r', 'h'):
      for side in ('0', '1'):
        out.append(p[gz + side + suf + '_w'][0, 0])
        out.append(p[gz + side + suf + '_b'][0])
    return out

  sv = jnp.stack(spack('2') + spack('4')
                 + [p['gated_w'][0, 0], p['gated_b'][0]]).reshape(1, 26)
  pw = jnp.concatenate([p['p1_w'], p['p2_w']], axis=1)
  pb = jnp.stack([p['p1_b'][0], p['p2_b'][0]]).reshape(1, 2)
  return (wq, bq, sv, pw, pb, p['l1_w'], p['l1_b'].reshape(1, 64),
          p['l2_w'], p['l2_b'].reshape(1, 32), p['l3_w'],
          p['l3_b'].reshape(1, 1))


def _tc_in_specs():
  return [
      pl.BlockSpec((G * RPE, EMBP), lambda i, ids, qts: (i, 0)),
      pl.BlockSpec((IDF_R, 128), lambda i, ids, qts: (0, 0)),
      pl.BlockSpec(memory_space=pltpu.MemorySpace.HBM),
      pl.BlockSpec((12, 5, 5), lambda i, ids, qts: (0, 0, 0)),
      pl.BlockSpec((12, 5, 1), lambda i, ids, qts: (0, 0, 0)),
      pl.BlockSpec((1, 26), lambda i, ids, qts: (0, 0)),
      pl.BlockSpec((5, 2), lambda i, ids, qts: (0, 0)),
      pl.BlockSpec((1, 2), lambda i, ids, qts: (0, 0)),
      pl.BlockSpec((120, 64), lambda i, ids, qts: (0, 0)),
      pl.BlockSpec((1, 64), lambda i, ids, qts: (0, 0)),
      pl.BlockSpec((64, 32), lambda i, ids, qts: (0, 0)),
      pl.BlockSpec((1, 32), lambda i, ids, qts: (0, 0)),
      pl.BlockSpec((32, 1), lambda i, ids, qts: (0, 0)),
      pl.BlockSpec((1, 1), lambda i, ids, qts: (0, 0)),
  ]


def kernel(qrl_token, doc_token, doc_ids, word_embedding, docs_adj,
           idf_table, params):
  qt = qrl_token.astype(jnp.int32)
  dt = doc_token.astype(jnp.int32)
  ids = doc_ids.astype(jnp.int32)
  fill = (lax.broadcasted_iota(jnp.int32, (B, 7), 0) * 7
          + lax.broadcasted_iota(jnp.int32, (B, 7), 1))
  tok = jnp.concatenate([qt, fill[:, 0:3], dt, fill[:, 3:7]],
                        axis=1).reshape(-1)

  def _pad_body(src_ref, dst_ref):
    dst_ref[:, 0:EMB] = src_ref[...]

  table_pad = pl.pallas_call(
      _pad_body,
      grid=(NROW // PAD_R,),
      in_specs=[pl.BlockSpec((PAD_R, EMB), lambda i: (i, 0))],
      out_specs=pl.BlockSpec((PAD_R, EMBP), lambda i: (i, 0)),
      out_shape=jax.ShapeDtypeStruct((NROW, EMBP), jnp.float32),
  )(word_embedding)
  rows = _sc_gather(table_pad, tok)

  idft = jnp.pad(idf_table, (0, IDF_R * 128 - idf_table.shape[0])
                 ).reshape(IDF_R, 128)

  out = pl.pallas_call(
      _tc_body,
      grid_spec=pltpu.PrefetchScalarGridSpec(
          num_scalar_prefetch=2,
          grid=(NB,),
          in_specs=_tc_in_specs(),
          out_specs=pl.BlockSpec((G, 1, 1), lambda i, ids, qts: (i, 0, 0)),
          scratch_shapes=[
              pltpu.VMEM((G, D, D), jnp.float32),
              pltpu.SemaphoreType.DMA((G,)),
          ],
      ),
      out_shape=jax.ShapeDtypeStruct((B, 1, 1), jnp.float32),
  )(ids, qt, rows, idft, docs_adj, *_pack(params))
  return out.reshape(B, 1)


# speedup vs baseline: 3.2522x; 1.0001x over previous
import functools

import jax
import jax.numpy as jnp
from jax import lax
from jax.experimental import pallas as pl
from jax.experimental.pallas import tpu as pltpu
from jax.experimental.pallas import tpu_sc as plsc

B = 64
Q = 5
D = 300
EMB = 300
EMBP = 384
PAD_R = 5000
NROW = 100000
K1 = 240
K2 = 192
TOPK = 40
RPE = 312
TOT = B * RPE
NW = 32
PER_W = TOT // NW
CH = 104
NCH = PER_W // CH
IDF_R = 784
G = 16
NB = B // G
BIG = 1e30


def _sc_gather(table_pad, tok_flat):
  mesh = plsc.VectorSubcoreMesh(core_axis_name="c", subcore_axis_name="s")

  @functools.partial(
      pl.kernel,
      mesh=mesh,
      out_type=jax.ShapeDtypeStruct((TOT, EMBP), jnp.float32),
      scratch_types=[
          pltpu.VMEM((CH,), jnp.int32),
          pltpu.VMEM((CH, EMBP), jnp.float32),
          pltpu.SemaphoreType.DMA,
      ],
  )
  def gather_kernel(table_hbm, tok_hbm, out_hbm, idx_v, rows_v, sem):
    c = lax.axis_index("c")
    s = lax.axis_index("s")
    wid = s * 2 + c
    base = wid * PER_W

    def chunk(i, carry):
      off = base + i * CH
      pltpu.sync_copy(tok_hbm.at[pl.ds(off, CH)], idx_v)
      pltpu.async_copy(table_hbm.at[idx_v], rows_v, sem).wait()
      pltpu.sync_copy(rows_v, out_hbm.at[pl.ds(off, CH)])
      return carry

    lax.fori_loop(0, NCH, chunk, 0)

  return gather_kernel(table_pad, tok_flat)


def _tc_body(ids_ref, qtok_ref, rows_ref, idft_ref, adj_hbm, wq_ref, bq_ref,
             sv_ref, pw_ref, pb_ref, l1w_ref, l1b_ref, l2w_ref, l2b_ref,
             l3w_ref, l3b_ref, out_ref, adjbuf, sem):
  f32 = jnp.float32
  i = pl.program_id(0)

  copies = []
  for g in range(G):
    cp = pltpu.make_async_copy(adj_hbm.at[ids_ref[i * G + g]],
                               adjbuf.at[g], sem.at[g])
    cp.start()
    copies.append(cp)

  def sig(x):
    return 1.0 / (1.0 + jnp.exp(-x))

  def lin_t(idx, x_t):
    return lax.dot_general(wq_ref[idx], x_t, (((0,), (0,)), ((), ())),
                           preferred_element_type=f32) + bq_ref[idx]

  def gated_q(i0, x_t, a_t):
    z = sig(lin_t(i0, a_t) + lin_t(i0 + 1, x_t))
    r = sig(lin_t(i0 + 2, a_t) + lin_t(i0 + 3, x_t))
    h = jnp.maximum(lin_t(i0 + 4, a_t) + lin_t(i0 + 5, r * x_t), 0.0)
    return h * z + x_t * (1.0 - z)

  def gated_s(o, x, a):
    sv = lambda j: sv_ref[0, o + j]
    z = sig(sv(0) * a + sv(1) + sv(2) * x + sv(3))
    r = sig(sv(4) * a + sv(5) + sv(6) * x + sv(7))
    h = jnp.maximum(sv(8) * a + sv(9) + sv(10) * (r * x) + sv(11), 0.0)
    return h * z + x * (1.0 - z)

  ei = lax.broadcasted_iota(jnp.int32, (D, D), 0)
  di = lax.broadcasted_iota(jnp.int32, (D, D), 1)
  ident = jnp.where(ei == di, 1.0, 0.0)
  ones_row = jnp.full((1, D), 1.0, f32)

  def topmask(s_row, kk):
    s_col = lax.dot_general(ident, s_row, (((1,), (1,)), ((), ())),
                            preferred_element_type=f32)
    Gm = jnp.where((s_col > s_row) | ((s_col == s_row) & (ei < di)), 1.0, 0.0)
    rank = lax.dot_general(ones_row, Gm, (((1,), (0,)), ((), ())),
                           preferred_element_type=f32)
    return rank < (kk - 0.5)

  V0s = []
  for g in range(G):
    qe = rows_ref[g * RPE + 0:g * RPE + Q, 0:EMB]
    dm = rows_ref[g * RPE + 8:g * RPE + 8 + D, 0:EMB]
    copies[g].wait()
    A = adjbuf[g]

    def amul(x_t):
      return lax.dot_general(x_t, A, (((1,), (1,)), ((), ())),
                             preferred_element_type=f32)

    dn = jnp.sqrt(jnp.sum(dm * dm, axis=1, keepdims=True)) + 1e-9
    qn = jnp.sqrt(jnp.sum(qe * qe, axis=1, keepdims=True)) + 1e-9
    feat_t = lax.dot_general(qe / qn, dm / dn, (((1,), (1,)), ((), ())),
                             preferred_element_type=f32)

    rep1_t = gated_q(0, feat_t, amul(feat_t))
    x2_t = lax.dot_general(pw_ref[:, 0:1], rep1_t, (((0,), (0,)), ((), ())),
                           preferred_element_type=f32) + pb_ref[0, 0]
    s_t = gated_s(0, x2_t, amul(x2_t))
    mask1 = topmask(s_t, K1)
    t1 = jnp.where(mask1, jnp.tanh(s_t), 0.0)
    xg_t = t1 * rep1_t
    att1m = jnp.where(mask1, xg_t, -BIG)
    rep3_t = gated_q(6, xg_t, amul(xg_t))
    x4_t = lax.dot_general(pw_ref[:, 1:2], rep3_t, (((0,), (0,)), ((), ())),
                           preferred_element_type=f32) + pb_ref[0, 1]
    s4_t = gated_s(12, x4_t, amul(jnp.where(mask1, x4_t, 0.0)))
    s4m = jnp.where(mask1, s4_t, -BIG)
    mask4 = topmask(s4m, K2)
    t4 = jnp.where(mask4, jnp.tanh(s4_t), 0.0)
    att2m = jnp.where(mask4, t4 * rep3_t, -BIG)
    V0s.append(feat_t)
    V0s.append(att1m)
    V0s.append(att2m)

  V0 = jnp.concatenate(V0s, axis=0)
  NR = 15 * G
  kcol = lax.broadcasted_iota(jnp.int32, (1, TOPK), 1)
  lane = lax.broadcasted_iota(jnp.int32, (NR, D), 1)
  big_i = jnp.int32(1 << 30)

  def step(k, carry):
    V, T = carry
    m = jnp.max(V, axis=1, keepdims=True)
    T = T + m * jnp.where(kcol == k, 1.0, 0.0)
    cand = jnp.where(V == m, lane, big_i)
    pmin = jnp.min(cand, axis=1, keepdims=True)
    V = jnp.where(cand == pmin, -BIG, V)
    return (V, T)

  _, T = lax.fori_loop(0, TOPK, step, (V0, jnp.zeros((NR, TOPK), f32)))

  lane128 = lax.broadcasted_iota(jnp.int32, (1, 128), 1)
  q5 = lax.broadcasted_iota(jnp.int32, (1, Q), 1)

  for g in range(G):
    Tg = T[15 * g:15 * g + 15]
    attx = jnp.concatenate([Tg[0:5], Tg[5:10], Tg[10:15]], axis=1)
    r1 = jnp.maximum(
        lax.dot_general(attx, l1w_ref[...], (((1,), (0,)), ((), ())),
                        preferred_element_type=f32) + l1b_ref[...], 0.0)
    r2 = jnp.maximum(
        lax.dot_general(r1, l2w_ref[...], (((1,), (0,)), ((), ())),
                        preferred_element_type=f32) + l2b_ref[...], 0.0)
    r3 = lax.dot_general(r2, l3w_ref[...], (((1,), (0,)), ((), ())),
                         preferred_element_type=f32) + l3b_ref[0, 0]

    b = i * G + g
    idf = jnp.zeros((1, Q), f32)
    for qi in range(Q):
      tok = qtok_ref[b, qi]
      rowv = idft_ref[pl.ds(tok // 128, 1), :]
      val = jnp.sum(jnp.where(lane128 == tok % 128, rowv, 0.0),
                    axis=1, keepdims=True)
      idf = idf + val * jnp.where(q5 == qi, 1.0, 0.0)
    gv = sv_ref[0, 24] * idf + sv_ref[0, 25]
    ge = jnp.exp(gv - jnp.max(gv, axis=1, keepdims=True))
    gw = ge / jnp.sum(ge, axis=1, keepdims=True)
    res = lax.dot_general(gw, r3, (((1,), (0,)), ((), ())),
                          preferred_element_type=f32)
    out_ref[g] = res


def _pack(p):
  qnames = ['z0', 'z1', 'r0', 'r1', 'h0', 'h1',
            'z03', 'z13', 'r03', 'r13', 'h03', 'h13']
  wq = jnp.stack([p[n + '_w'] for n in qnames])
  bq = jnp.stack([p[n + '_b'] for n in qnames])[:, :, None]

  def spack(suf):
    out = []
    for gz in ('z', 'r', 'h'):
      for side in ('0', '1'):
        out.append(p[gz + side + suf + '_w'][0, 0])
        out.append(p[gz + side + suf + '_b'][0])
    return out

  sv = jnp.stack(spack('2') + spack('4')
                 + [p['gated_w'][0, 0], p['gated_b'][0]]).reshape(1, 26)
  pw = jnp.concatenate([p['p1_w'], p['p2_w']], axis=1)
  pb = jnp.stack([p['p1_b'][0], p['p2_b'][0]]).reshape(1, 2)
  return (wq, bq, sv, pw, pb, p['l1_w'], p['l1_b'].reshape(1, 64),
          p['l2_w'], p['l2_b'].reshape(1, 32), p['l3_w'],
          p['l3_b'].reshape(1, 1))


def _tc_in_specs():
  return [
      pl.BlockSpec((G * RPE, EMBP), lambda i, ids, qts: (i, 0)),
      pl.BlockSpec((IDF_R, 128), lambda i, ids, qts: (0, 0)),
      pl.BlockSpec(memory_space=pltpu.MemorySpace.HBM),
      pl.BlockSpec((12, 5, 5), lambda i, ids, qts: (0, 0, 0)),
      pl.BlockSpec((12, 5, 1), lambda i, ids, qts: (0, 0, 0)),
      pl.BlockSpec((1, 26), lambda i, ids, qts: (0, 0)),
      pl.BlockSpec((5, 2), lambda i, ids, qts: (0, 0)),
      pl.BlockSpec((1, 2), lambda i, ids, qts: (0, 0)),
      pl.BlockSpec((120, 64), lambda i, ids, qts: (0, 0)),
      pl.BlockSpec((1, 64), lambda i, ids, qts: (0, 0)),
      pl.BlockSpec((64, 32), lambda i, ids, qts: (0, 0)),
      pl.BlockSpec((1, 32), lambda i, ids, qts: (0, 0)),
      pl.BlockSpec((32, 1), lambda i, ids, qts: (0, 0)),
      pl.BlockSpec((1, 1), lambda i, ids, qts: (0, 0)),
  ]


def kernel(qrl_token, doc_token, doc_ids, word_embedding, docs_adj,
           idf_table, params):
  qt = qrl_token.astype(jnp.int32)
  dt = doc_token.astype(jnp.int32)
  ids = doc_ids.astype(jnp.int32)
  fill = (lax.broadcasted_iota(jnp.int32, (B, 7), 0) * 7
          + lax.broadcasted_iota(jnp.int32, (B, 7), 1))
  tok = jnp.concatenate([qt, fill[:, 0:3], dt, fill[:, 3:7]],
                        axis=1).reshape(-1)

  def _pad_body(src_ref, dst_ref):
    dst_ref[:, 0:EMB] = src_ref[...]

  table_pad = pl.pallas_call(
      _pad_body,
      grid=(NROW // PAD_R,),
      in_specs=[pl.BlockSpec((PAD_R, EMB), lambda i: (i, 0))],
      out_specs=pl.BlockSpec((PAD_R, EMBP), lambda i: (i, 0)),
      out_shape=jax.ShapeDtypeStruct((NROW, EMBP), jnp.float32),
      compiler_params=pltpu.CompilerParams(
          dimension_semantics=("parallel",)),
  )(word_embedding)
  rows = _sc_gather(table_pad, tok)

  idft = jnp.pad(idf_table, (0, IDF_R * 128 - idf_table.shape[0])
                 ).reshape(IDF_R, 128)

  out = pl.pallas_call(
      _tc_body,
      grid_spec=pltpu.PrefetchScalarGridSpec(
          num_scalar_prefetch=2,
          grid=(NB,),
          in_specs=_tc_in_specs(),
          out_specs=pl.BlockSpec((G, 1, 1), lambda i, ids, qts: (i, 0, 0)),
          scratch_shapes=[
              pltpu.VMEM((G, D, D), jnp.float32),
              pltpu.SemaphoreType.DMA((G,)),
          ],
      ),
      out_shape=jax.ShapeDtypeStruct((B, 1, 1), jnp.float32),
      compiler_params=pltpu.CompilerParams(
          dimension_semantics=("parallel",)),
  )(ids, qt, rows, idft, docs_adj, *_pack(params))
  return out.reshape(B, 1)


# speedup vs baseline: 3.3486x; 1.0296x over previous
import functools

import jax
import jax.numpy as jnp
from jax import lax
from jax.experimental import pallas as pl
from jax.experimental.pallas import tpu as pltpu
from jax.experimental.pallas import tpu_sc as plsc

B = 64
Q = 5
D = 300
EMB = 300
EMBP = 384
PAD_R = 5000
NROW = 100000
K1 = 240
K2 = 192
TOPK = 40
RPE = 312
TOT = B * RPE
NW = 32
PER_W = TOT // NW
CH = 104
NCH = PER_W // CH
IDF_R = 784
G = 32
NB = B // G
BIG = 1e30


def _sc_gather(table_pad, tok_flat):
  mesh = plsc.VectorSubcoreMesh(core_axis_name="c", subcore_axis_name="s")

  @functools.partial(
      pl.kernel,
      mesh=mesh,
      out_type=jax.ShapeDtypeStruct((TOT, EMBP), jnp.float32),
      scratch_types=[
          pltpu.VMEM((CH,), jnp.int32),
          pltpu.VMEM((CH, EMBP), jnp.float32),
          pltpu.SemaphoreType.DMA,
      ],
  )
  def gather_kernel(table_hbm, tok_hbm, out_hbm, idx_v, rows_v, sem):
    c = lax.axis_index("c")
    s = lax.axis_index("s")
    wid = s * 2 + c
    base = wid * PER_W

    def chunk(i, carry):
      off = base + i * CH
      pltpu.sync_copy(tok_hbm.at[pl.ds(off, CH)], idx_v)
      pltpu.async_copy(table_hbm.at[idx_v], rows_v, sem).wait()
      pltpu.sync_copy(rows_v, out_hbm.at[pl.ds(off, CH)])
      return carry

    lax.fori_loop(0, NCH, chunk, 0)

  return gather_kernel(table_pad, tok_flat)


def _tc_body(ids_ref, qtok_ref, rows_ref, idft_ref, adj_hbm, wq_ref, bq_ref,
             sv_ref, pw_ref, pb_ref, l1w_ref, l1b_ref, l2w_ref, l2b_ref,
             l3w_ref, l3b_ref, out_ref, adjbuf, sem):
  f32 = jnp.float32
  i = pl.program_id(0)

  copies = []
  for g in range(G):
    cp = pltpu.make_async_copy(adj_hbm.at[ids_ref[i * G + g]],
                               adjbuf.at[g], sem.at[g])
    cp.start()
    copies.append(cp)

  def sig(x):
    return 1.0 / (1.0 + jnp.exp(-x))

  def lin_t(idx, x_t):
    return lax.dot_general(wq_ref[idx], x_t, (((0,), (0,)), ((), ())),
                           preferred_element_type=f32) + bq_ref[idx]

  def gated_q(i0, x_t, a_t):
    z = sig(lin_t(i0, a_t) + lin_t(i0 + 1, x_t))
    r = sig(lin_t(i0 + 2, a_t) + lin_t(i0 + 3, x_t))
    h = jnp.maximum(lin_t(i0 + 4, a_t) + lin_t(i0 + 5, r * x_t), 0.0)
    return h * z + x_t * (1.0 - z)

  def gated_s(o, x, a):
    sv = lambda j: sv_ref[0, o + j]
    z = sig(sv(0) * a + sv(1) + sv(2) * x + sv(3))
    r = sig(sv(4) * a + sv(5) + sv(6) * x + sv(7))
    h = jnp.maximum(sv(8) * a + sv(9) + sv(10) * (r * x) + sv(11), 0.0)
    return h * z + x * (1.0 - z)

  ei = lax.broadcasted_iota(jnp.int32, (D, D), 0)
  di = lax.broadcasted_iota(jnp.int32, (D, D), 1)
  ident = jnp.where(ei == di, 1.0, 0.0)
  ones_row = jnp.full((1, D), 1.0, f32)

  def topmask(s_row, kk):
    s_col = lax.dot_general(ident, s_row, (((1,), (1,)), ((), ())),
                            preferred_element_type=f32)
    Gm = jnp.where((s_col > s_row) | ((s_col == s_row) & (ei < di)), 1.0, 0.0)
    rank = lax.dot_general(ones_row, Gm, (((1,), (0,)), ((), ())),
                           preferred_element_type=f32)
    return rank < (kk - 0.5)

  V0s = []
  for g in range(G):
    qe = rows_ref[g * RPE + 0:g * RPE + Q, 0:EMB]
    dm = rows_ref[g * RPE + 8:g * RPE + 8 + D, 0:EMB]
    copies[g].wait()
    A = adjbuf[g]

    def amul(x_t):
      return lax.dot_general(x_t, A, (((1,), (1,)), ((), ())),
                             preferred_element_type=f32)

    dn = jnp.sqrt(jnp.sum(dm * dm, axis=1, keepdims=True)) + 1e-9
    qn = jnp.sqrt(jnp.sum(qe * qe, axis=1, keepdims=True)) + 1e-9
    feat_t = lax.dot_general(qe / qn, dm / dn, (((1,), (1,)), ((), ())),
                             preferred_element_type=f32)

    rep1_t = gated_q(0, feat_t, amul(feat_t))
    x2_t = lax.dot_general(pw_ref[:, 0:1], rep1_t, (((0,), (0,)), ((), ())),
                           preferred_element_type=f32) + pb_ref[0, 0]
    s_t = gated_s(0, x2_t, amul(x2_t))
    mask1 = topmask(s_t, K1)
    t1 = jnp.where(mask1, jnp.tanh(s_t), 0.0)
    xg_t = t1 * rep1_t
    att1m = jnp.where(mask1, xg_t, -BIG)
    rep3_t = gated_q(6, xg_t, amul(xg_t))
    x4_t = lax.dot_general(pw_ref[:, 1:2], rep3_t, (((0,), (0,)), ((), ())),
                           preferred_element_type=f32) + pb_ref[0, 1]
    s4_t = gated_s(12, x4_t, amul(jnp.where(mask1, x4_t, 0.0)))
    s4m = jnp.where(mask1, s4_t, -BIG)
    mask4 = topmask(s4m, K2)
    t4 = jnp.where(mask4, jnp.tanh(s4_t), 0.0)
    att2m = jnp.where(mask4, t4 * rep3_t, -BIG)
    V0s.append(feat_t)
    V0s.append(att1m)
    V0s.append(att2m)

  V0 = jnp.concatenate(V0s, axis=0)
  NR = 15 * G
  kcol = lax.broadcasted_iota(jnp.int32, (1, TOPK), 1)
  lane = lax.broadcasted_iota(jnp.int32, (NR, D), 1)
  big_i = jnp.int32(1 << 30)

  def step(k, carry):
    V, T = carry
    m = jnp.max(V, axis=1, keepdims=True)
    T = T + m * jnp.where(kcol == k, 1.0, 0.0)
    cand = jnp.where(V == m, lane, big_i)
    pmin = jnp.min(cand, axis=1, keepdims=True)
    V = jnp.where(cand == pmin, -BIG, V)
    return (V, T)

  _, T = lax.fori_loop(0, TOPK, step, (V0, jnp.zeros((NR, TOPK), f32)))

  lane128 = lax.broadcasted_iota(jnp.int32, (1, 128), 1)
  q5 = lax.broadcasted_iota(jnp.int32, (1, Q), 1)

  for g in range(G):
    Tg = T[15 * g:15 * g + 15]
    attx = jnp.concatenate([Tg[0:5], Tg[5:10], Tg[10:15]], axis=1)
    r1 = jnp.maximum(
        lax.dot_general(attx, l1w_ref[...], (((1,), (0,)), ((), ())),
                        preferred_element_type=f32) + l1b_ref[...], 0.0)
    r2 = jnp.maximum(
        lax.dot_general(r1, l2w_ref[...], (((1,), (0,)), ((), ())),
                        preferred_element_type=f32) + l2b_ref[...], 0.0)
    r3 = lax.dot_general(r2, l3w_ref[...], (((1,), (0,)), ((), ())),
                         preferred_element_type=f32) + l3b_ref[0, 0]

    b = i * G + g
    idf = jnp.zeros((1, Q), f32)
    for qi in range(Q):
      tok = qtok_ref[b, qi]
      rowv = idft_ref[pl.ds(tok // 128, 1), :]
      val = jnp.sum(jnp.where(lane128 == tok % 128, rowv, 0.0),
                    axis=1, keepdims=True)
      idf = idf + val * jnp.where(q5 == qi, 1.0, 0.0)
    gv = sv_ref[0, 24] * idf + sv_ref[0, 25]
    ge = jnp.exp(gv - jnp.max(gv, axis=1, keepdims=True))
    gw = ge / jnp.sum(ge, axis=1, keepdims=True)
    res = lax.dot_general(gw, r3, (((1,), (0,)), ((), ())),
                          preferred_element_type=f32)
    out_ref[g] = res


def _pack(p):
  qnames = ['z0', 'z1', 'r0', 'r1', 'h0', 'h1',
            'z03', 'z13', 'r03', 'r13', 'h03', 'h13']
  wq = jnp.stack([p[n + '_w'] for n in qnames])
  bq = jnp.stack([p[n + '_b'] for n in qnames])[:, :, None]

  def spack(suf):
    out = []
    for gz in ('z', 'r', 'h'):
      for side in ('0', '1'):
        out.append(p[gz + side + suf + '_w'][0, 0])
        out.append(p[gz + side + suf + '_b'][0])
    return out

  sv = jnp.stack(spack('2') + spack('4')
                 + [p['gated_w'][0, 0], p['gated_b'][0]]).reshape(1, 26)
  pw = jnp.concatenate([p['p1_w'], p['p2_w']], axis=1)
  pb = jnp.stack([p['p1_b'][0], p['p2_b'][0]]).reshape(1, 2)
  return (wq, bq, sv, pw, pb, p['l1_w'], p['l1_b'].reshape(1, 64),
          p['l2_w'], p['l2_b'].reshape(1, 32), p['l3_w'],
          p['l3_b'].reshape(1, 1))


def _tc_in_specs():
  return [
      pl.BlockSpec((G * RPE, EMBP), lambda i, ids, qts: (i, 0)),
      pl.BlockSpec((IDF_R, 128), lambda i, ids, qts: (0, 0)),
      pl.BlockSpec(memory_space=pltpu.MemorySpace.HBM),
      pl.BlockSpec((12, 5, 5), lambda i, ids, qts: (0, 0, 0)),
      pl.BlockSpec((12, 5, 1), lambda i, ids, qts: (0, 0, 0)),
      pl.BlockSpec((1, 26), lambda i, ids, qts: (0, 0)),
      pl.BlockSpec((5, 2), lambda i, ids, qts: (0, 0)),
      pl.BlockSpec((1, 2), lambda i, ids, qts: (0, 0)),
      pl.BlockSpec((120, 64), lambda i, ids, qts: (0, 0)),
      pl.BlockSpec((1, 64), lambda i, ids, qts: (0, 0)),
      pl.BlockSpec((64, 32), lambda i, ids, qts: (0, 0)),
      pl.BlockSpec((1, 32), lambda i, ids, qts: (0, 0)),
      pl.BlockSpec((32, 1), lambda i, ids, qts: (0, 0)),
      pl.BlockSpec((1, 1), lambda i, ids, qts: (0, 0)),
  ]


def kernel(qrl_token, doc_token, doc_ids, word_embedding, docs_adj,
           idf_table, params):
  qt = qrl_token.astype(jnp.int32)
  dt = doc_token.astype(jnp.int32)
  ids = doc_ids.astype(jnp.int32)
  fill = (lax.broadcasted_iota(jnp.int32, (B, 7), 0) * 7
          + lax.broadcasted_iota(jnp.int32, (B, 7), 1))
  tok = jnp.concatenate([qt, fill[:, 0:3], dt, fill[:, 3:7]],
                        axis=1).reshape(-1)

  def _pad_body(src_ref, dst_ref):
    dst_ref[:, 0:EMB] = src_ref[...]

  table_pad = pl.pallas_call(
      _pad_body,
      grid=(NROW // PAD_R,),
      in_specs=[pl.BlockSpec((PAD_R, EMB), lambda i: (i, 0))],
      out_specs=pl.BlockSpec((PAD_R, EMBP), lambda i: (i, 0)),
      out_shape=jax.ShapeDtypeStruct((NROW, EMBP), jnp.float32),
      compiler_params=pltpu.CompilerParams(
          dimension_semantics=("parallel",)),
  )(word_embedding)
  rows = _sc_gather(table_pad, tok)

  idft = jnp.pad(idf_table, (0, IDF_R * 128 - idf_table.shape[0])
                 ).reshape(IDF_R, 128)

  out = pl.pallas_call(
      _tc_body,
      grid_spec=pltpu.PrefetchScalarGridSpec(
          num_scalar_prefetch=2,
          grid=(NB,),
          in_specs=_tc_in_specs(),
          out_specs=pl.BlockSpec((G, 1, 1), lambda i, ids, qts: (i, 0, 0)),
          scratch_shapes=[
              pltpu.VMEM((G, D, D), jnp.float32),
              pltpu.SemaphoreType.DMA((G,)),
          ],
      ),
      out_shape=jax.ShapeDtypeStruct((B, 1, 1), jnp.float32),
      compiler_params=pltpu.CompilerParams(
          dimension_semantics=("parallel",)),
  )(ids, qt, rows, idft, docs_adj, *_pack(params))
  return out.reshape(B, 1)


# speedup vs baseline: 3.3512x; 1.0008x over previous
import functools

import jax
import jax.numpy as jnp
from jax import lax
from jax.experimental import pallas as pl
from jax.experimental.pallas import tpu as pltpu
from jax.experimental.pallas import tpu_sc as plsc

B = 64
Q = 5
D = 300
EMB = 300
EMBP = 384
PAD_R = 5000
NROW = 100000
K1 = 240
K2 = 192
TOPK = 40
RPE = 312
TOT = B * RPE
NW = 32
PER_W = TOT // NW
CH = 104
NCH = PER_W // CH
IDF_R = 784
G = 32
NB = B // G
BIG = 1e30


def _sc_gather(table_pad, tok_flat):
  mesh = plsc.VectorSubcoreMesh(core_axis_name="c", subcore_axis_name="s")

  @functools.partial(
      pl.kernel,
      mesh=mesh,
      out_type=jax.ShapeDtypeStruct((TOT, EMBP), jnp.float32),
      scratch_types=[
          pltpu.VMEM((CH,), jnp.int32),
          pltpu.VMEM((CH, EMBP), jnp.float32),
          pltpu.SemaphoreType.DMA,
      ],
  )
  def gather_kernel(table_hbm, tok_hbm, out_hbm, idx_v, rows_v, sem):
    c = lax.axis_index("c")
    s = lax.axis_index("s")
    wid = s * 2 + c
    base = wid * PER_W

    def chunk(i, carry):
      off = base + i * CH
      pltpu.sync_copy(tok_hbm.at[pl.ds(off, CH)], idx_v)
      pltpu.async_copy(table_hbm.at[idx_v], rows_v, sem).wait()
      pltpu.sync_copy(rows_v, out_hbm.at[pl.ds(off, CH)])
      return carry

    lax.fori_loop(0, NCH, chunk, 0)

  return gather_kernel(table_pad, tok_flat)


def _tc_body(ids_ref, qtok_ref, rows_ref, idft_ref, adj_hbm, wq_ref, bq_ref,
             sv_ref, pw_ref, pb_ref, l1w_ref, l1b_ref, l2w_ref, l2b_ref,
             l3w_ref, l3b_ref, out_ref, adjbuf, sem):
  f32 = jnp.float32
  i = pl.program_id(0)

  copies = []
  for g in range(G):
    cp = pltpu.make_async_copy(adj_hbm.at[ids_ref[i * G + g]],
                               adjbuf.at[g], sem.at[g])
    cp.start()
    copies.append(cp)

  def sig(x):
    return 1.0 / (1.0 + jnp.exp(-x))

  def lin_t(idx, x_t):
    return lax.dot_general(wq_ref[idx], x_t, (((0,), (0,)), ((), ())),
                           preferred_element_type=f32) + bq_ref[idx]

  def gated_q(i0, x_t, a_t):
    z = sig(lin_t(i0, a_t) + lin_t(i0 + 1, x_t))
    r = sig(lin_t(i0 + 2, a_t) + lin_t(i0 + 3, x_t))
    h = jnp.maximum(lin_t(i0 + 4, a_t) + lin_t(i0 + 5, r * x_t), 0.0)
    return h * z + x_t * (1.0 - z)

  def gated_s(o, x, a):
    sv = lambda j: sv_ref[0, o + j]
    z = sig(sv(0) * a + sv(1) + sv(2) * x + sv(3))
    r = sig(sv(4) * a + sv(5) + sv(6) * x + sv(7))
    h = jnp.maximum(sv(8) * a + sv(9) + sv(10) * (r * x) + sv(11), 0.0)
    return h * z + x * (1.0 - z)

  ei = lax.broadcasted_iota(jnp.int32, (D, D), 0)
  di = lax.broadcasted_iota(jnp.int32, (D, D), 1)
  ident = jnp.where(ei == di, 1.0, 0.0)
  ones_row = jnp.full((1, D), 1.0, f32)

  def topmask(s_row, kk):
    s_col = lax.dot_general(ident, s_row, (((1,), (1,)), ((), ())),
                            preferred_element_type=f32)
    Gm = jnp.where((s_col > s_row) | ((s_col == s_row) & (ei < di)), 1.0, 0.0)
    rank = lax.dot_general(ones_row, Gm, (((1,), (0,)), ((), ())),
                           preferred_element_type=f32)
    return rank < (kk - 0.5)

  V0s = []
  for g in range(G):
    qe = rows_ref[g * RPE + 0:g * RPE + Q, 0:EMB]
    dm = rows_ref[g * RPE + 8:g * RPE + 8 + D, 0:EMB]
    copies[g].wait()
    A = adjbuf[g]

    def amul(x_t):
      return lax.dot_general(x_t, A, (((1,), (1,)), ((), ())),
                             preferred_element_type=f32)

    dn = jnp.sqrt(jnp.sum(dm * dm, axis=1, keepdims=True)) + 1e-9
    qn = jnp.sqrt(jnp.sum(qe * qe, axis=1, keepdims=True)) + 1e-9
    feat_t = lax.dot_general(qe / qn, dm / dn, (((1,), (1,)), ((), ())),
                             preferred_element_type=f32)

    rep1_t = gated_q(0, feat_t, amul(feat_t))
    x2_t = lax.dot_general(pw_ref[:, 0:1], rep1_t, (((0,), (0,)), ((), ())),
                           preferred_element_type=f32) + pb_ref[0, 0]
    s_t = gated_s(0, x2_t, amul(x2_t))
    mask1 = topmask(s_t, K1)
    t1 = jnp.where(mask1, jnp.tanh(s_t), 0.0)
    xg_t = t1 * rep1_t
    att1m = jnp.where(mask1, xg_t, -BIG)
    rep3_t = gated_q(6, xg_t, amul(xg_t))
    x4_t = lax.dot_general(pw_ref[:, 1:2], rep3_t, (((0,), (0,)), ((), ())),
                           preferred_element_type=f32) + pb_ref[0, 1]
    s4_t = gated_s(12, x4_t, amul(jnp.where(mask1, x4_t, 0.0)))
    s4m = jnp.where(mask1, s4_t, -BIG)
    mask4 = topmask(s4m, K2)
    t4 = jnp.where(mask4, jnp.tanh(s4_t), 0.0)
    att2m = jnp.where(mask4, t4 * rep3_t, -BIG)
    V0s.append(feat_t)
    V0s.append(att1m)
    V0s.append(att2m)

  V0 = jnp.concatenate(V0s, axis=0)
  NR = 15 * G
  kcol = lax.broadcasted_iota(jnp.int32, (NR, TOPK), 1)

  def step(k, carry):
    V, T, kc = carry
    m = jnp.max(V, axis=1, keepdims=True)
    eq = V == m
    cnt = jnp.sum(jnp.where(eq, 1, 0), axis=1, keepdims=True)
    fill = (kcol >= kc) & (kcol < kc + cnt)
    T = T + m * jnp.where(fill, 1.0, 0.0)
    V = jnp.where(eq, -BIG, V)
    return (V, T, kc + cnt)

  _, T, _ = lax.fori_loop(
      0, TOPK, step,
      (V0, jnp.zeros((NR, TOPK), f32), jnp.zeros((NR, 1), jnp.int32)))

  lane128 = lax.broadcasted_iota(jnp.int32, (1, 128), 1)
  q5 = lax.broadcasted_iota(jnp.int32, (1, Q), 1)

  for g in range(G):
    Tg = T[15 * g:15 * g + 15]
    attx = jnp.concatenate([Tg[0:5], Tg[5:10], Tg[10:15]], axis=1)
    r1 = jnp.maximum(
        lax.dot_general(attx, l1w_ref[...], (((1,), (0,)), ((), ())),
                        preferred_element_type=f32) + l1b_ref[...], 0.0)
    r2 = jnp.maximum(
        lax.dot_general(r1, l2w_ref[...], (((1,), (0,)), ((), ())),
                        preferred_element_type=f32) + l2b_ref[...], 0.0)
    r3 = lax.dot_general(r2, l3w_ref[...], (((1,), (0,)), ((), ())),
                         preferred_element_type=f32) + l3b_ref[0, 0]

    b = i * G + g
    idf = jnp.zeros((1, Q), f32)
    for qi in range(Q):
      tok = qtok_ref[b, qi]
      rowv = idft_ref[pl.ds(tok // 128, 1), :]
      val = jnp.sum(jnp.where(lane128 == tok % 128, rowv, 0.0),
                    axis=1, keepdims=True)
      idf = idf + val * jnp.where(q5 == qi, 1.0, 0.0)
    gv = sv_ref[0, 24] * idf + sv_ref[0, 25]
    ge = jnp.exp(gv - jnp.max(gv, axis=1, keepdims=True))
    gw = ge / jnp.sum(ge, axis=1, keepdims=True)
    res = lax.dot_general(gw, r3, (((1,), (0,)), ((), ())),
                          preferred_element_type=f32)
    out_ref[g] = res


def _pack(p):
  qnames = ['z0', 'z1', 'r0', 'r1', 'h0', 'h1',
            'z03', 'z13', 'r03', 'r13', 'h03', 'h13']
  wq = jnp.stack([p[n + '_w'] for n in qnames])
  bq = jnp.stack([p[n + '_b'] for n in qnames])[:, :, None]

  def spack(suf):
    out = []
    for gz in ('z', 'r', 'h'):
      for side in ('0', '1'):
        out.append(p[gz + side + suf + '_w'][0, 0])
        out.append(p[gz + side + suf + '_b'][0])
    return out

  sv = jnp.stack(spack('2') + spack('4')
                 + [p['gated_w'][0, 0], p['gated_b'][0]]).reshape(1, 26)
  pw = jnp.concatenate([p['p1_w'], p['p2_w']], axis=1)
  pb = jnp.stack([p['p1_b'][0], p['p2_b'][0]]).reshape(1, 2)
  return (wq, bq, sv, pw, pb, p['l1_w'], p['l1_b'].reshape(1, 64),
          p['l2_w'], p['l2_b'].reshape(1, 32), p['l3_w'],
          p['l3_b'].reshape(1, 1))


def _tc_in_specs():
  return [
      pl.BlockSpec((G * RPE, EMBP), lambda i, ids, qts: (i, 0)),
      pl.BlockSpec((IDF_R, 128), lambda i, ids, qts: (0, 0)),
      pl.BlockSpec(memory_space=pltpu.MemorySpace.HBM),
      pl.BlockSpec((12, 5, 5), lambda i, ids, qts: (0, 0, 0)),
      pl.BlockSpec((12, 5, 1), lambda i, ids, qts: (0, 0, 0)),
      pl.BlockSpec((1, 26), lambda i, ids, qts: (0, 0)),
      pl.BlockSpec((5, 2), lambda i, ids, qts: (0, 0)),
      pl.BlockSpec((1, 2), lambda i, ids, qts: (0, 0)),
      pl.BlockSpec((120, 64), lambda i, ids, qts: (0, 0)),
      pl.BlockSpec((1, 64), lambda i, ids, qts: (0, 0)),
      pl.BlockSpec((64, 32), lambda i, ids, qts: (0, 0)),
      pl.BlockSpec((1, 32), lambda i, ids, qts: (0, 0)),
      pl.BlockSpec((32, 1), lambda i, ids, qts: (0, 0)),
      pl.BlockSpec((1, 1), lambda i, ids, qts: (0, 0)),
  ]


def kernel(qrl_token, doc_token, doc_ids, word_embedding, docs_adj,
           idf_table, params):
  qt = qrl_token.astype(jnp.int32)
  dt = doc_token.astype(jnp.int32)
  ids = doc_ids.astype(jnp.int32)
  fill = (lax.broadcasted_iota(jnp.int32, (B, 7), 0) * 7
          + lax.broadcasted_iota(jnp.int32, (B, 7), 1))
  tok = jnp.concatenate([qt, fill[:, 0:3], dt, fill[:, 3:7]],
                        axis=1).reshape(-1)

  def _pad_body(src_ref, dst_ref):
    dst_ref[:, 0:EMB] = src_ref[...]

  table_pad = pl.pallas_call(
      _pad_body,
      grid=(NROW // PAD_R,),
      in_specs=[pl.BlockSpec((PAD_R, EMB), lambda i: (i, 0))],
      out_specs=pl.BlockSpec((PAD_R, EMBP), lambda i: (i, 0)),
      out_shape=jax.ShapeDtypeStruct((NROW, EMBP), jnp.float32),
      compiler_params=pltpu.CompilerParams(
          dimension_semantics=("parallel",)),
  )(word_embedding)
  rows = _sc_gather(table_pad, tok)

  idft = jnp.pad(idf_table, (0, IDF_R * 128 - idf_table.shape[0])
                 ).reshape(IDF_R, 128)

  out = pl.pallas_call(
      _tc_body,
      grid_spec=pltpu.PrefetchScalarGridSpec(
          num_scalar_prefetch=2,
          grid=(NB,),
          in_specs=_tc_in_specs(),
          out_specs=pl.BlockSpec((G, 1, 1), lambda i, ids, qts: (i, 0, 0)),
          scratch_shapes=[
              pltpu.VMEM((G, D, D), jnp.float32),
              pltpu.SemaphoreType.DMA((G,)),
          ],
      ),
      out_shape=jax.ShapeDtypeStruct((B, 1, 1), jnp.float32),
      compiler_params=pltpu.CompilerParams(
          dimension_semantics=("parallel",)),
  )(ids, qt, rows, idft, docs_adj, *_pack(params))
  return out.reshape(B, 1)
